# Initial kernel scaffold; baseline (speedup 1.0000x reference)
import functools

import jax
import jax.numpy as jnp
from jax import lax
from jax.experimental import pallas as pl
from jax.experimental.pallas import tpu as pltpu
from jax.experimental.pallas import tpu_sc as plsc

N_USERS = 50000
N_ITEMS = 50000
N = N_USERS + N_ITEMS
E = 1600000
N_LAYERS = 3
D = 32

NC = 2
NS = 16
HALF = N // NC
CHUNK = 1024
MICRO = 128
NMICRO = CHUNK // MICRO
NCHUNK = 98
E_PAD = NS * NCHUNK * CHUNK
SPAD_PER_TILE = 3200
SPAD_ROWS = NS * SPAD_PER_TILE
DUMMY = HALF
OUT_PER_TILE = HALF // NS


def _spmm_body(rows_hbm, cols_hbm, vals_hbm, ego_hbm, out_hbm,
               cols2, idx2, rows_f, vals_f, gath, zbuf, sem, spad):
    c = lax.axis_index("c")
    s = lax.axis_index("s")
    base = c * HALF

    zv = jnp.zeros((16,), jnp.float32)
    for r in range(MICRO):
        zbuf[r, pl.ds(0, 16)] = zv
        zbuf[r, pl.ds(16, 16)] = zv

    def zero_body(k, _):
        pltpu.sync_copy(zbuf, spad.at[pl.ds(s * SPAD_PER_TILE + k * MICRO,
                                            MICRO)])
        return 0
    lax.fori_loop(0, SPAD_PER_TILE // MICRO, zero_body, 0)
    plsc.subcore_barrier()

    eye16 = lax.iota(jnp.int32, 16)

    def chunk_body(ch, _):
        gchunk = s * NCHUNK + ch
        ebase = gchunk * CHUNK
        mbase = gchunk * NMICRO
        pltpu.sync_copy(rows_hbm.at[pl.ds(ebase, CHUNK)], rows_f)
        pltpu.sync_copy(vals_hbm.at[pl.ds(ebase, CHUNK)], vals_f)
        pltpu.sync_copy(cols_hbm.at[pl.ds(mbase, NMICRO)], cols2)

        handles = []
        for j in range(NMICRO):
            handles.append(pltpu.async_copy(
                ego_hbm.at[cols2.at[j]],
                gath.at[pl.ds(j * MICRO, MICRO)], sem))
        for h in handles:
            h.wait()

        for j in range(NMICRO):
            for l in range(8):
                r = rows_f[pl.ds(j * MICRO + l * 16, 16)]
                rl = r - base
                ok = (rl >= 0) & (rl < HALF)
                idx2[j, pl.ds(l * 16, 16)] = jnp.where(ok, rl, DUMMY)

        def scale_body(grp, _):
            b = grp * 16
            eid = eye16 + b
            vv = vals_f[pl.ds(b, 16)]
            for d in range(D):
                dvec = jnp.full((16,), d, jnp.int32)
                x = plsc.load_gather(gath, [eid, dvec])
                plsc.store_scatter(gath, [eid, dvec], x * vv)
            return 0
        lax.fori_loop(0, CHUNK // 16, scale_body, 0)

        for j in range(NMICRO):
            pltpu.sync_copy(gath.at[pl.ds(j * MICRO, MICRO)],
                            spad.at[idx2.at[j]], add=True)
        return 0
    lax.fori_loop(0, NCHUNK, chunk_body, 0)
    plsc.subcore_barrier()

    r0 = s * OUT_PER_TILE
    o0 = base + r0

    def out_body(k, _):
        pltpu.sync_copy(spad.at[pl.ds(r0 + k * MICRO, MICRO)],
                        gath.at[pl.ds(0, MICRO)])
        pltpu.sync_copy(gath.at[pl.ds(0, MICRO)],
                        out_hbm.at[pl.ds(o0 + k * MICRO, MICRO)])
        return 0
    nfull = OUT_PER_TILE // MICRO
    rem = OUT_PER_TILE - nfull * MICRO
    lax.fori_loop(0, nfull, out_body, 0)
    pltpu.sync_copy(spad.at[pl.ds(r0 + nfull * MICRO, rem)],
                    gath.at[pl.ds(0, rem)])
    pltpu.sync_copy(gath.at[pl.ds(0, rem)],
                    out_hbm.at[pl.ds(o0 + nfull * MICRO, rem)])


_spmm = pl.kernel(
    _spmm_body,
    out_type=jax.ShapeDtypeStruct((N, D), jnp.float32),
    mesh=plsc.VectorSubcoreMesh(core_axis_name="c", subcore_axis_name="s"),
    scratch_types=[
        pltpu.VMEM((NMICRO, MICRO), jnp.int32),
        pltpu.VMEM((NMICRO, MICRO), jnp.int32),
        pltpu.VMEM((CHUNK,), jnp.int32),
        pltpu.VMEM((CHUNK,), jnp.float32),
        pltpu.VMEM((CHUNK, D), jnp.float32),
        pltpu.VMEM((MICRO, D), jnp.float32),
        pltpu.SemaphoreType.DMA,
        pltpu.VMEM_SHARED((SPAD_ROWS, D), jnp.float32),
    ],
)


def _enc_body(text_ref, w1_ref, w2_ref, id_ref, out_ref):
    h = jnp.maximum(jnp.dot(text_ref[...], w1_ref[...],
                            preferred_element_type=jnp.float32), 0.0)
    created = jnp.dot(h, w2_ref[...], preferred_element_type=jnp.float32)
    out_ref[...] = jnp.concatenate([id_ref[...], created], axis=1)


def _encode(text, w1, w2, id_emb, bm=500):
    n = text.shape[0]
    grid = n // bm
    return pl.pallas_call(
        _enc_body,
        grid=(grid,),
        in_specs=[
            pl.BlockSpec((bm, text.shape[1]), lambda i: (i, 0)),
            pl.BlockSpec(w1.shape, lambda i: (0, 0)),
            pl.BlockSpec(w2.shape, lambda i: (0, 0)),
            pl.BlockSpec((bm, id_emb.shape[1]), lambda i: (i, 0)),
        ],
        out_specs=pl.BlockSpec((bm, D), lambda i: (i, 0)),
        out_shape=jax.ShapeDtypeStruct((n, D), jnp.float32),
    )(text, w1, w2, id_emb)


def _leaky(x):
    return jnp.where(x >= 0, x, 0.01 * x)


def _dense_body(side_ref, ego_ref, gcw_ref, gcb_ref, biw_ref, bib_ref,
                new_ref, nrm_ref):
    sde = side_ref[...]
    ego = ego_ref[...]
    sum_e = _leaky(jnp.dot(sde, gcw_ref[...],
                           preferred_element_type=jnp.float32) + gcb_ref[...])
    bi = _leaky(jnp.dot(ego * sde, biw_ref[...],
                        preferred_element_type=jnp.float32) + bib_ref[...])
    ne = sum_e + bi
    nrm = jnp.sqrt(jnp.sum(ne * ne, axis=1, keepdims=True))
    new_ref[...] = ne
    nrm_ref[...] = ne / jnp.maximum(nrm, 1e-12)


def _dense_update(side, ego, gcw, gcb, biw, bib, bm=1000):
    grid = N // bm
    full = lambda a: pl.BlockSpec(a.shape, lambda i: (0, 0))
    return pl.pallas_call(
        _dense_body,
        grid=(grid,),
        in_specs=[
            pl.BlockSpec((bm, D), lambda i: (i, 0)),
            pl.BlockSpec((bm, D), lambda i: (i, 0)),
            full(gcw), full(gcb), full(biw), full(bib),
        ],
        out_specs=[pl.BlockSpec((bm, D), lambda i: (i, 0)),
                   pl.BlockSpec((bm, D), lambda i: (i, 0))],
        out_shape=[jax.ShapeDtypeStruct((N, D), jnp.float32),
                   jax.ShapeDtypeStruct((N, D), jnp.float32)],
    )(side, ego, gcw, gcb, biw, bib)


def kernel(adj_indices, adj_values, user_name_embs, sent_embs, user_emb,
           item_emb, u_w1, u_w2, i_w1, i_w2, gc_w, gc_b, bi_w, bi_b):
    ego_u = _encode(user_name_embs, u_w1, u_w2, user_emb)
    ego_i = _encode(sent_embs, i_w1, i_w2, item_emb)
    ego = jnp.concatenate([ego_u, ego_i], axis=0)

    rows = adj_indices[0].astype(jnp.int32)
    cols = adj_indices[1].astype(jnp.int32)
    pad = E_PAD - E
    rows_p = jnp.concatenate([rows, jnp.full((pad,), N, jnp.int32)])
    cols_p = jnp.concatenate([cols, jnp.zeros((pad,), jnp.int32)])
    vals_p = jnp.concatenate([adj_values,
                              jnp.zeros((pad,), jnp.float32)])
    cols2 = cols_p.reshape(E_PAD // MICRO, MICRO)

    outs = [ego]
    for i in range(N_LAYERS):
        side = _spmm(rows_p, cols2, vals_p, ego)
        ego, nrm = _dense_update(side, ego, gc_w[i], gc_b[i].reshape(1, D),
                                 bi_w[i], bi_b[i].reshape(1, D))
        outs.append(nrm)
    all_e = jnp.concatenate(outs, axis=1)
    return all_e[:N_USERS], all_e[N_USERS:]


# speedup vs baseline: 2.6204x; 2.6204x over previous
import functools

import jax
import jax.numpy as jnp
from jax import lax
from jax.experimental import pallas as pl
from jax.experimental.pallas import tpu as pltpu
from jax.experimental.pallas import tpu_sc as plsc

N_USERS = 50000
N_ITEMS = 50000
N = N_USERS + N_ITEMS
E = 1600000
N_LAYERS = 3
D = 32

NC = 2
NS = 16
QUART = N // 4
CHUNK = 1024
MICRO = 128
NMICRO = CHUNK // MICRO
NCHUNK = 98
E_PAD = NS * NCHUNK * CHUNK
SPAD_PER_TILE = 1664
SPAD_ROWS = NS * SPAD_PER_TILE
DUMMY = QUART
OUT_PER_TILE = 1568


def _spmm_body(rows_hbm, cols_hbm, vals_hbm, ego_hbm, out_hbm,
               cols2, idx2, rows_f, vals_f, gath, zbuf, sem, spad):
    c = lax.axis_index("c")
    s = lax.axis_index("s")

    zv = jnp.zeros((16,), jnp.float32)
    for r in range(MICRO):
        zbuf[r, pl.ds(0, 16)] = zv
        zbuf[r, pl.ds(16, 16)] = zv

    for p in range(2):
        base = (c * 2 + p) * QUART

        def zero_body(k, _):
            pltpu.sync_copy(zbuf,
                            spad.at[pl.ds(s * SPAD_PER_TILE + k * MICRO,
                                          MICRO)])
            return 0
        lax.fori_loop(0, SPAD_PER_TILE // MICRO, zero_body, 0)
        plsc.subcore_barrier()

        def chunk_body(ch, _):
            gchunk = s * NCHUNK + ch
            ebase = gchunk * CHUNK
            mbase = gchunk * NMICRO
            pltpu.sync_copy(rows_hbm.at[pl.ds(ebase, CHUNK)], rows_f)
            pltpu.sync_copy(vals_hbm.at[pl.ds(ebase, CHUNK)], vals_f)
            pltpu.sync_copy(cols_hbm.at[pl.ds(mbase, NMICRO)], cols2)

            handles = []
            for j in range(NMICRO):
                handles.append(pltpu.async_copy(
                    ego_hbm.at[cols2.at[j]],
                    gath.at[pl.ds(j * MICRO, MICRO)], sem))
            for h in handles:
                h.wait()

            for j in range(NMICRO):
                for l in range(8):
                    r = rows_f[pl.ds(j * MICRO + l * 16, 16)]
                    rl = r - base
                    ok = (rl >= 0) & (rl < QUART)
                    idx2[j, pl.ds(l * 16, 16)] = jnp.where(ok, rl, DUMMY)

            def scale_body(grp, _):
                b = grp * 16
                vv = vals_f[pl.ds(b, 16)]
                for k in range(16):
                    e = b + k
                    v = vv[k]
                    gath[e, pl.ds(0, 16)] = gath[e, pl.ds(0, 16)] * v
                    gath[e, pl.ds(16, 16)] = gath[e, pl.ds(16, 16)] * v
                return 0
            lax.fori_loop(0, CHUNK // 16, scale_body, 0)

            for j in range(NMICRO):
                pltpu.sync_copy(gath.at[pl.ds(j * MICRO, MICRO)],
                                spad.at[idx2.at[j]], add=True)
            return 0
        lax.fori_loop(0, NCHUNK, chunk_body, 0)
        plsc.subcore_barrier()

        r0 = s * OUT_PER_TILE
        o0 = base + r0

        def out_body(k, _):
            pltpu.sync_copy(spad.at[pl.ds(r0 + k * MICRO, MICRO)],
                            gath.at[pl.ds(0, MICRO)])
            pltpu.sync_copy(gath.at[pl.ds(0, MICRO)],
                            out_hbm.at[pl.ds(o0 + k * MICRO, MICRO)])
            return 0
        nfull = jnp.where(s < NS - 1, 12, 11)
        lax.fori_loop(0, nfull, out_body, 0)

        @pl.when(s < NS - 1)
        def _():
            pltpu.sync_copy(spad.at[pl.ds(r0 + 12 * MICRO, 32)],
                            gath.at[pl.ds(0, 32)])
            pltpu.sync_copy(gath.at[pl.ds(0, 32)],
                            out_hbm.at[pl.ds(o0 + 12 * MICRO, 32)])

        @pl.when(s == NS - 1)
        def _():
            pltpu.sync_copy(spad.at[pl.ds(r0 + 11 * MICRO, 72)],
                            gath.at[pl.ds(0, 72)])
            pltpu.sync_copy(gath.at[pl.ds(0, 72)],
                            out_hbm.at[pl.ds(o0 + 11 * MICRO, 72)])
        plsc.subcore_barrier()


_spmm = pl.kernel(
    _spmm_body,
    out_type=jax.ShapeDtypeStruct((N, D), jnp.float32),
    mesh=plsc.VectorSubcoreMesh(core_axis_name="c", subcore_axis_name="s"),
    scratch_types=[
        pltpu.VMEM((NMICRO, MICRO), jnp.int32),
        pltpu.VMEM((NMICRO, MICRO), jnp.int32),
        pltpu.VMEM((CHUNK,), jnp.int32),
        pltpu.VMEM((CHUNK,), jnp.float32),
        pltpu.VMEM((CHUNK, D), jnp.float32),
        pltpu.VMEM((MICRO, D), jnp.float32),
        pltpu.SemaphoreType.DMA,
        pltpu.VMEM_SHARED((SPAD_ROWS, D), jnp.float32),
    ],
    compiler_params=pltpu.CompilerParams(use_tc_tiling_on_sc=False),
)


def _enc_body(text_ref, w1_ref, w2_ref, id_ref, out_ref):
    h = jnp.maximum(jnp.dot(text_ref[...], w1_ref[...],
                            preferred_element_type=jnp.float32), 0.0)
    created = jnp.dot(h, w2_ref[...], preferred_element_type=jnp.float32)
    out_ref[...] = jnp.concatenate([id_ref[...], created], axis=1)


def _encode(text, w1, w2, id_emb, bm=400):
    n = text.shape[0]
    grid = n // bm
    return pl.pallas_call(
        _enc_body,
        grid=(grid,),
        in_specs=[
            pl.BlockSpec((bm, text.shape[1]), lambda i: (i, 0)),
            pl.BlockSpec(w1.shape, lambda i: (0, 0)),
            pl.BlockSpec(w2.shape, lambda i: (0, 0)),
            pl.BlockSpec((bm, id_emb.shape[1]), lambda i: (i, 0)),
        ],
        out_specs=pl.BlockSpec((bm, D), lambda i: (i, 0)),
        out_shape=jax.ShapeDtypeStruct((n, D), jnp.float32),
    )(text, w1, w2, id_emb)


def _leaky(x):
    return jnp.where(x >= 0, x, 0.01 * x)


def _dense_body(side_ref, ego_ref, gcw_ref, gcb_ref, biw_ref, bib_ref,
                new_ref, nrm_ref):
    sde = side_ref[...]
    ego = ego_ref[...]
    sum_e = _leaky(jnp.dot(sde, gcw_ref[...],
                           preferred_element_type=jnp.float32) + gcb_ref[...])
    bi = _leaky(jnp.dot(ego * sde, biw_ref[...],
                        preferred_element_type=jnp.float32) + bib_ref[...])
    ne = sum_e + bi
    nrm = jnp.sqrt(jnp.sum(ne * ne, axis=1, keepdims=True))
    new_ref[...] = ne
    nrm_ref[...] = ne / jnp.maximum(nrm, 1e-12)


def _dense_update(side, ego, gcw, gcb, biw, bib, bm=1000):
    grid = N // bm
    full = lambda a: pl.BlockSpec(a.shape, lambda i: (0, 0))
    return pl.pallas_call(
        _dense_body,
        grid=(grid,),
        in_specs=[
            pl.BlockSpec((bm, D), lambda i: (i, 0)),
            pl.BlockSpec((bm, D), lambda i: (i, 0)),
            full(gcw), full(gcb), full(biw), full(bib),
        ],
        out_specs=[pl.BlockSpec((bm, D), lambda i: (i, 0)),
                   pl.BlockSpec((bm, D), lambda i: (i, 0))],
        out_shape=[jax.ShapeDtypeStruct((N, D), jnp.float32),
                   jax.ShapeDtypeStruct((N, D), jnp.float32)],
    )(side, ego, gcw, gcb, biw, bib)


def kernel(adj_indices, adj_values, user_name_embs, sent_embs, user_emb,
           item_emb, u_w1, u_w2, i_w1, i_w2, gc_w, gc_b, bi_w, bi_b):
    ego_u = _encode(user_name_embs, u_w1, u_w2, user_emb)
    ego_i = _encode(sent_embs, i_w1, i_w2, item_emb)
    ego = jnp.concatenate([ego_u, ego_i], axis=0)

    rows = adj_indices[0].astype(jnp.int32)
    cols = adj_indices[1].astype(jnp.int32)
    pad = E_PAD - E
    rows_p = jnp.concatenate([rows, jnp.full((pad,), N, jnp.int32)])
    cols_p = jnp.concatenate([cols, jnp.zeros((pad,), jnp.int32)])
    vals_p = jnp.concatenate([adj_values,
                              jnp.zeros((pad,), jnp.float32)])
    cols2 = cols_p.reshape(E_PAD // MICRO, MICRO)

    outs = [ego]
    for i in range(N_LAYERS):
        side = _spmm(rows_p, cols2, vals_p, ego)
        ego, nrm = _dense_update(side, ego, gc_w[i], gc_b[i].reshape(1, D),
                                 bi_w[i], bi_b[i].reshape(1, D))
        outs.append(nrm)
    all_e = jnp.concatenate(outs, axis=1)
    return all_e[:N_USERS], all_e[N_USERS:]
